# Initial kernel scaffold; baseline (speedup 1.0000x reference)
import jax
import jax.numpy as jnp
from jax.experimental import pallas as pl


def kernel(outputs, targets):
    raise NotImplementedError("write your pallas kernel here")


# speedup vs baseline: 23.9018x; 23.9018x over previous
import jax
import jax.numpy as jnp
from jax.experimental import pallas as pl
from jax.experimental.pallas import tpu as pltpu

N = 16384
K = 16
KA = 24
TI = 4096
TJ = 2048
NI = N // TI
NJ = N // TJ
TP = 2048
NP = N // TP


def _prep_body(b_ref, out_ref):
    b = b_ref[...]
    bt = b.T
    bn = jnp.sum(bt * bt, axis=0, keepdims=True)
    out_ref[...] = jnp.concatenate(
        [-2.0 * bt, bn, jnp.zeros((KA - K - 1, TP), jnp.float32)], axis=0)


def _fold_min(val):
    parts = [val[:, k * 128:(k + 1) * 128] for k in range(TJ // 128)]
    lvl1 = [jnp.minimum(parts[t], parts[t + 1]) for t in range(0, len(parts), 2)]
    m = lvl1[0]
    for p in lvl1[1:]:
        m = jnp.minimum(m, p)
    return m


def _body(a_ref, bt_ref, out_ref, minacc_ref):
    i = pl.program_id(0)
    j = pl.program_id(1)
    a = a_ref[...]
    bt = bt_ref[...]
    val = jax.lax.dot_general(a, bt, (((1,), (0,)), ((), ())),
                              preferred_element_type=jnp.float32)
    m = _fold_min(val)

    @pl.when(j == 0)
    def _():
        minacc_ref[...] = m

    @pl.when(j > 0)
    def _():
        minacc_ref[...] = jnp.minimum(minacc_ref[...], m)

    @pl.when(j == NJ - 1)
    def _():
        an = jnp.sum(a[:, :K] * a[:, :K], axis=1, keepdims=True)
        row = jnp.min(minacc_ref[...], axis=1, keepdims=True) + an
        out_ref[0, 0, 0] = jnp.sum(row) * (1.0 / N)


@jax.jit
def kernel(outputs, targets):
    bt_aug = pl.pallas_call(
        _prep_body,
        grid=(NP,),
        in_specs=[pl.BlockSpec((TP, K), lambda j: (j, 0))],
        out_specs=pl.BlockSpec((KA, TP), lambda j: (0, j)),
        out_shape=jax.ShapeDtypeStruct((KA, N), jnp.float32),
    )(targets)

    a_aug = jnp.concatenate(
        [outputs,
         jnp.ones((N, 1), jnp.float32),
         jnp.zeros((N, KA - K - 1), jnp.float32)], axis=1)

    res = pl.pallas_call(
        _body,
        grid=(NI, NJ),
        in_specs=[
            pl.BlockSpec((TI, KA), lambda i, j: (i, 0)),
            pl.BlockSpec((KA, TJ), lambda i, j: (0, j)),
        ],
        out_specs=pl.BlockSpec(
            (1, 1, 1), lambda i, j: (i, 0, 0), memory_space=pltpu.SMEM),
        out_shape=jax.ShapeDtypeStruct((NI, 1, 1), jnp.float32),
        scratch_shapes=[pltpu.VMEM((TI, 128), jnp.float32)],
        compiler_params=pltpu.CompilerParams(
            dimension_semantics=("parallel", "arbitrary")),
    )(a_aug, bt_aug)
    return jnp.sum(res)


# speedup vs baseline: 24.5774x; 1.0283x over previous
import jax
import jax.numpy as jnp
from jax.experimental import pallas as pl
from jax.experimental.pallas import tpu as pltpu

N = 16384
K = 16
KA = 24
TI = 512
TJC = 2048
NI = N // TI
NJC = N // TJC
TP = 2048
NP = N // TP


def _prep_body(b_ref, out_ref):
    b = b_ref[...]
    bt = b.T
    bn = jnp.sum(bt * bt, axis=0, keepdims=True)
    out_ref[...] = jnp.concatenate(
        [-2.0 * bt, bn, jnp.zeros((KA - K - 1, TP), jnp.float32)], axis=0)


def _fold_min(val):
    parts = [val[:, k * 128:(k + 1) * 128] for k in range(TJC // 128)]
    while len(parts) > 1:
        nxt = [jnp.minimum(parts[t], parts[t + 1])
               for t in range(0, len(parts) - 1, 2)]
        if len(parts) % 2:
            nxt.append(parts[-1])
        parts = nxt
    return parts[0]


def _body(a_ref, bt_ref, out_ref):
    a = a_ref[...]
    acc = None
    for c in range(NJC):
        bt = bt_ref[:, c * TJC:(c + 1) * TJC]
        val = jax.lax.dot_general(a, bt, (((1,), (0,)), ((), ())),
                                  preferred_element_type=jnp.float32)
        m = _fold_min(val)
        acc = m if acc is None else jnp.minimum(acc, m)

    an = jnp.sum(a[:, :K] * a[:, :K], axis=1, keepdims=True)
    row = jnp.min(acc, axis=1, keepdims=True) + an
    out_ref[0, 0, 0] = jnp.sum(row) * (1.0 / N)


@jax.jit
def kernel(outputs, targets):
    bt_aug = pl.pallas_call(
        _prep_body,
        grid=(NP,),
        in_specs=[pl.BlockSpec((TP, K), lambda j: (j, 0))],
        out_specs=pl.BlockSpec((KA, TP), lambda j: (0, j)),
        out_shape=jax.ShapeDtypeStruct((KA, N), jnp.float32),
    )(targets)

    a_aug = jnp.concatenate(
        [outputs,
         jnp.ones((N, 1), jnp.float32),
         jnp.zeros((N, KA - K - 1), jnp.float32)], axis=1)

    res = pl.pallas_call(
        _body,
        grid=(NI,),
        in_specs=[
            pl.BlockSpec((TI, KA), lambda i: (i, 0)),
            pl.BlockSpec((KA, N), lambda i: (0, 0)),
        ],
        out_specs=pl.BlockSpec(
            (1, 1, 1), lambda i: (i, 0, 0), memory_space=pltpu.SMEM),
        out_shape=jax.ShapeDtypeStruct((NI, 1, 1), jnp.float32),
        compiler_params=pltpu.CompilerParams(
            dimension_semantics=("parallel",)),
    )(a_aug, bt_aug)
    return jnp.sum(res)


# speedup vs baseline: 25.3806x; 1.0327x over previous
import jax
import jax.numpy as jnp
from jax.experimental import pallas as pl
from jax.experimental.pallas import tpu as pltpu

N = 16384
K = 16
KA = 24
TI = 512
TJC = 2048
NI = N // TI
NJC = N // TJC


def _fold_min(val):
    parts = [val[:, k * 128:(k + 1) * 128] for k in range(TJC // 128)]
    while len(parts) > 1:
        nxt = [jnp.minimum(parts[t], parts[t + 1])
               for t in range(0, len(parts) - 1, 2)]
        if len(parts) % 2:
            nxt.append(parts[-1])
        parts = nxt
    return parts[0]


def _body(a_ref, t_ref, out_ref, bt_ref):
    i = pl.program_id(0)

    @pl.when(i == 0)
    def _():
        t = t_ref[...]
        bt = t.T
        bn = jnp.sum(bt * bt, axis=0, keepdims=True)
        bt_ref[...] = jnp.concatenate(
            [-2.0 * bt, bn, jnp.zeros((KA - K - 1, N), jnp.float32)], axis=0)

    a = a_ref[...]
    a_aug = jnp.concatenate(
        [a, jnp.ones((TI, 1), jnp.float32),
         jnp.zeros((TI, KA - K - 1), jnp.float32)], axis=1)

    acc = None
    for c in range(NJC):
        bt = bt_ref[:, c * TJC:(c + 1) * TJC]
        val = jax.lax.dot_general(a_aug, bt, (((1,), (0,)), ((), ())),
                                  preferred_element_type=jnp.float32)
        m = _fold_min(val)
        acc = m if acc is None else jnp.minimum(acc, m)

    an = jnp.sum(a * a, axis=1, keepdims=True)
    row = jnp.min(acc, axis=1, keepdims=True) + an
    s = jnp.sum(row) * (1.0 / N)

    @pl.when(i == 0)
    def _():
        out_ref[0, 0] = s

    @pl.when(i > 0)
    def _():
        out_ref[0, 0] += s


@jax.jit
def kernel(outputs, targets):
    res = pl.pallas_call(
        _body,
        grid=(NI,),
        in_specs=[
            pl.BlockSpec((TI, K), lambda i: (i, 0)),
            pl.BlockSpec((N, K), lambda i: (0, 0)),
        ],
        out_specs=pl.BlockSpec(
            (1, 1), lambda i: (0, 0), memory_space=pltpu.SMEM),
        out_shape=jax.ShapeDtypeStruct((1, 1), jnp.float32),
        scratch_shapes=[pltpu.VMEM((KA, N), jnp.float32)],
        compiler_params=pltpu.CompilerParams(
            dimension_semantics=("arbitrary",)),
    )(outputs, targets)
    return res[0, 0]


# speedup vs baseline: 25.3997x; 1.0008x over previous
import jax
import jax.numpy as jnp
from jax.experimental import pallas as pl
from jax.experimental.pallas import tpu as pltpu

N = 16384
K = 16
KA = 24
TI = 512
TJC = 4096
NI = N // TI
NJC = N // TJC


def _fold_min(val):
    parts = [val[:, k * 128:(k + 1) * 128] for k in range(TJC // 128)]
    while len(parts) > 1:
        nxt = [jnp.minimum(parts[t], parts[t + 1])
               for t in range(0, len(parts) - 1, 2)]
        if len(parts) % 2:
            nxt.append(parts[-1])
        parts = nxt
    return parts[0]


def _body(a_ref, t_ref, out_ref, bt_ref):
    i = pl.program_id(0)

    @pl.when(i == 0)
    def _():
        t = t_ref[...]
        bt = t.T
        bn = jnp.sum(bt * bt, axis=0, keepdims=True)
        bt_ref[...] = jnp.concatenate(
            [-2.0 * bt, bn, jnp.zeros((KA - K - 1, N), jnp.float32)], axis=0)

    a = a_ref[...]
    a_aug = jnp.concatenate(
        [a, jnp.ones((TI, 1), jnp.float32),
         jnp.zeros((TI, KA - K - 1), jnp.float32)], axis=1)

    acc = None
    for c in range(NJC):
        bt = bt_ref[:, c * TJC:(c + 1) * TJC]
        val = jax.lax.dot_general(a_aug, bt, (((1,), (0,)), ((), ())),
                                  preferred_element_type=jnp.float32)
        m = _fold_min(val)
        acc = m if acc is None else jnp.minimum(acc, m)

    an = jnp.sum(a * a, axis=1, keepdims=True)
    row = jnp.min(acc, axis=1, keepdims=True) + an
    s = jnp.sum(row) * (1.0 / N)

    @pl.when(i == 0)
    def _():
        out_ref[0, 0] = s

    @pl.when(i > 0)
    def _():
        out_ref[0, 0] += s


@jax.jit
def kernel(outputs, targets):
    res = pl.pallas_call(
        _body,
        grid=(NI,),
        in_specs=[
            pl.BlockSpec((TI, K), lambda i: (i, 0)),
            pl.BlockSpec((N, K), lambda i: (0, 0)),
        ],
        out_specs=pl.BlockSpec(
            (1, 1), lambda i: (0, 0), memory_space=pltpu.SMEM),
        out_shape=jax.ShapeDtypeStruct((1, 1), jnp.float32),
        scratch_shapes=[pltpu.VMEM((KA, N), jnp.float32)],
        compiler_params=pltpu.CompilerParams(
            dimension_semantics=("arbitrary",)),
    )(outputs, targets)
    return res[0, 0]


# speedup vs baseline: 26.3049x; 1.0356x over previous
import jax
import jax.numpy as jnp
from jax.experimental import pallas as pl
from jax.experimental.pallas import tpu as pltpu

N = 16384
K = 16
KA = 24
TI = 1024
TJC = 4096
NI = N // TI
NJC = N // TJC


def _fold_min(val):
    parts = [val[:, k * 128:(k + 1) * 128] for k in range(TJC // 128)]
    while len(parts) > 1:
        nxt = [jnp.minimum(parts[t], parts[t + 1])
               for t in range(0, len(parts) - 1, 2)]
        if len(parts) % 2:
            nxt.append(parts[-1])
        parts = nxt
    return parts[0]


def _body(a_ref, t_ref, out_ref, bt_ref):
    i = pl.program_id(0)

    @pl.when(i == 0)
    def _():
        t = t_ref[...]
        bt = t.T
        bn = jnp.sum(bt * bt, axis=0, keepdims=True)
        bt_ref[...] = jnp.concatenate(
            [-2.0 * bt, bn, jnp.zeros((KA - K - 1, N), jnp.float32)], axis=0)

    a = a_ref[...]
    a_aug = jnp.concatenate(
        [a, jnp.ones((TI, 1), jnp.float32),
         jnp.zeros((TI, KA - K - 1), jnp.float32)], axis=1)

    acc = None
    for c in range(NJC):
        bt = bt_ref[:, c * TJC:(c + 1) * TJC]
        val = jax.lax.dot_general(a_aug, bt, (((1,), (0,)), ((), ())),
                                  preferred_element_type=jnp.float32)
        m = _fold_min(val)
        acc = m if acc is None else jnp.minimum(acc, m)

    an = jnp.sum(a * a, axis=1, keepdims=True)
    row = jnp.min(acc, axis=1, keepdims=True) + an
    s = jnp.sum(row) * (1.0 / N)

    @pl.when(i == 0)
    def _():
        out_ref[0, 0] = s

    @pl.when(i > 0)
    def _():
        out_ref[0, 0] += s


@jax.jit
def kernel(outputs, targets):
    res = pl.pallas_call(
        _body,
        grid=(NI,),
        in_specs=[
            pl.BlockSpec((TI, K), lambda i: (i, 0)),
            pl.BlockSpec((N, K), lambda i: (0, 0)),
        ],
        out_specs=pl.BlockSpec(
            (1, 1), lambda i: (0, 0), memory_space=pltpu.SMEM),
        out_shape=jax.ShapeDtypeStruct((1, 1), jnp.float32),
        scratch_shapes=[pltpu.VMEM((KA, N), jnp.float32)],
        compiler_params=pltpu.CompilerParams(
            dimension_semantics=("arbitrary",)),
    )(outputs, targets)
    return res[0, 0]


# speedup vs baseline: 26.8116x; 1.0193x over previous
import jax
import jax.numpy as jnp
from jax.experimental import pallas as pl
from jax.experimental.pallas import tpu as pltpu

N = 16384
K = 16
KA = 24
TI = 2048
TJC = 4096
NI = N // TI
NJC = N // TJC


def _fold_min(val):
    parts = [val[:, k * 128:(k + 1) * 128] for k in range(TJC // 128)]
    while len(parts) > 1:
        nxt = [jnp.minimum(parts[t], parts[t + 1])
               for t in range(0, len(parts) - 1, 2)]
        if len(parts) % 2:
            nxt.append(parts[-1])
        parts = nxt
    return parts[0]


def _body(a_ref, t_ref, out_ref, bt_ref):
    i = pl.program_id(0)

    @pl.when(i == 0)
    def _():
        t = t_ref[...]
        bt = t.T
        bn = jnp.sum(bt * bt, axis=0, keepdims=True)
        bt_ref[...] = jnp.concatenate(
            [-2.0 * bt, bn, jnp.zeros((KA - K - 1, N), jnp.float32)], axis=0)

    a = a_ref[...]
    a_aug = jnp.concatenate(
        [a, jnp.ones((TI, 1), jnp.float32),
         jnp.zeros((TI, KA - K - 1), jnp.float32)], axis=1)

    acc = None
    for c in range(NJC):
        bt = bt_ref[:, c * TJC:(c + 1) * TJC]
        val = jax.lax.dot_general(a_aug, bt, (((1,), (0,)), ((), ())),
                                  preferred_element_type=jnp.float32)
        m = _fold_min(val)
        acc = m if acc is None else jnp.minimum(acc, m)

    an = jnp.sum(a * a, axis=1, keepdims=True)
    row = jnp.min(acc, axis=1, keepdims=True) + an
    s = jnp.sum(row) * (1.0 / N)

    @pl.when(i == 0)
    def _():
        out_ref[0, 0] = s

    @pl.when(i > 0)
    def _():
        out_ref[0, 0] += s


@jax.jit
def kernel(outputs, targets):
    res = pl.pallas_call(
        _body,
        grid=(NI,),
        in_specs=[
            pl.BlockSpec((TI, K), lambda i: (i, 0)),
            pl.BlockSpec((N, K), lambda i: (0, 0)),
        ],
        out_specs=pl.BlockSpec(
            (1, 1), lambda i: (0, 0), memory_space=pltpu.SMEM),
        out_shape=jax.ShapeDtypeStruct((1, 1), jnp.float32),
        scratch_shapes=[pltpu.VMEM((KA, N), jnp.float32)],
        compiler_params=pltpu.CompilerParams(
            dimension_semantics=("arbitrary",)),
    )(outputs, targets)
    return res[0, 0]
